# baseline (speedup 1.0000x reference)
import functools

import jax
import jax.numpy as jnp
from jax.experimental import pallas as pl
from jax.experimental.pallas import tpu as pltpu

EMBED_DIM = 768
NUM_EXPERTS = 64
TOP_K = 2
LOAD_BALANCE_WEIGHT = 0.01

TOKEN_BLOCK = 4096


def _router_kernel(x_ref, w_ref, b_ref, idx_ref, pk_ref, probs_ref, part_ref):
    logits = jax.lax.dot_general(
        x_ref[...], w_ref[...], (((1,), (1,)), ((), ())),
        preferred_element_type=jnp.float32) + b_ref[...]
    m = jnp.max(logits, axis=-1, keepdims=True)
    e = jnp.exp(logits - m)
    s = jnp.sum(e, axis=-1, keepdims=True)
    p = e / s
    probs_ref[...] = p

    iota = jax.lax.broadcasted_iota(jnp.int32, p.shape, 1)
    p1 = jnp.max(p, axis=-1, keepdims=True)
    i1 = jnp.min(jnp.where(p == p1, iota, NUM_EXPERTS), axis=-1, keepdims=True)
    pm = jnp.where(iota == i1, -1.0, p)
    p2 = jnp.max(pm, axis=-1, keepdims=True)
    i2 = jnp.min(jnp.where(pm == p2, iota, NUM_EXPERTS), axis=-1, keepdims=True)

    denom = p1 + p2
    p1n = p1 / denom
    p2n = p2 / denom
    idx_ref[...] = jnp.concatenate([i1, i2], axis=1)
    pk_ref[...] = jnp.concatenate([p1n, p2n], axis=1)

    usage_sum = jnp.sum(p, axis=0, keepdims=True)
    ent_sum = jnp.sum(p * jnp.log(p + 1e-8), keepdims=True).reshape(1, 1)
    conf_sum = jnp.sum(p1n, keepdims=True).reshape(1, 1)
    part_ref[...] = jnp.concatenate(
        [usage_sum, ent_sum, conf_sum, jnp.zeros((1, 62), jnp.float32)],
        axis=1).reshape(1, 1, 128)


def _stats_kernel(n_tokens, part_ref, usage_ref, loss_ref, ent_ref, conf_ref):
    part = part_ref[...].reshape(part_ref.shape[0], 128)
    inv_n = 1.0 / n_tokens
    u = jnp.sum(part[:, :NUM_EXPERTS], axis=0, keepdims=True) * inv_n
    usage_ref[...] = u
    t = 1.0 / NUM_EXPERTS
    kl = jnp.sum(t * (jnp.log(t) - jnp.log(u)), keepdims=True) / NUM_EXPERTS
    loss_ref[...] = kl.reshape(1, 1) * LOAD_BALANCE_WEIGHT
    ent_ref[...] = -(jnp.sum(part[:, NUM_EXPERTS:NUM_EXPERTS + 1],
                             keepdims=True).reshape(1, 1) * inv_n)
    conf_ref[...] = jnp.sum(part[:, NUM_EXPERTS + 1:NUM_EXPERTS + 2],
                            keepdims=True).reshape(1, 1) * inv_n


@jax.jit
def kernel(hidden_states, W, b):
    B, S, D = hidden_states.shape
    n = B * S
    x = hidden_states.reshape(n, D)
    b2 = b.reshape(1, NUM_EXPERTS)
    T = TOKEN_BLOCK
    nsteps = n // T

    idx, pk, probs, part = pl.pallas_call(
        _router_kernel,
        grid=(nsteps,),
        in_specs=[
            pl.BlockSpec((T, D), lambda i: (i, 0)),
            pl.BlockSpec((NUM_EXPERTS, D), lambda i: (0, 0)),
            pl.BlockSpec((1, NUM_EXPERTS), lambda i: (0, 0)),
        ],
        out_specs=(
            pl.BlockSpec((T, TOP_K), lambda i: (i, 0)),
            pl.BlockSpec((T, TOP_K), lambda i: (i, 0)),
            pl.BlockSpec((T, NUM_EXPERTS), lambda i: (i, 0)),
            pl.BlockSpec((1, 1, 128), lambda i: (i, 0, 0)),
        ),
        out_shape=(
            jax.ShapeDtypeStruct((n, TOP_K), jnp.int32),
            jax.ShapeDtypeStruct((n, TOP_K), jnp.float32),
            jax.ShapeDtypeStruct((n, NUM_EXPERTS), jnp.float32),
            jax.ShapeDtypeStruct((nsteps, 1, 128), jnp.float32),
        ),
        compiler_params=pltpu.CompilerParams(
            dimension_semantics=("parallel",)),
    )(x, W, b2)

    usage, loss, ent, conf = pl.pallas_call(
        functools.partial(_stats_kernel, float(n)),
        out_shape=(
            jax.ShapeDtypeStruct((1, NUM_EXPERTS), jnp.float32),
            jax.ShapeDtypeStruct((1, 1), jnp.float32),
            jax.ShapeDtypeStruct((1, 1), jnp.float32),
            jax.ShapeDtypeStruct((1, 1), jnp.float32),
        ),
    )(part)

    return (idx.reshape(B, S, TOP_K),
            pk.reshape(B, S, TOP_K),
            probs.reshape(B, S, NUM_EXPERTS),
            loss.reshape(()),
            ent.reshape(()),
            usage.reshape(NUM_EXPERTS),
            conf.reshape(()))


# speedup vs baseline: 1.0039x; 1.0039x over previous
import functools

import jax
import jax.numpy as jnp
from jax.experimental import pallas as pl
from jax.experimental.pallas import tpu as pltpu

EMBED_DIM = 768
NUM_EXPERTS = 64
TOP_K = 2
LOAD_BALANCE_WEIGHT = 0.01

TOKEN_BLOCK = 4096


def _router_kernel(x_ref, w_ref, b_ref, idx_ref, pk_ref, probs_ref, part_ref):
    logits = jax.lax.dot_general(
        x_ref[...], w_ref[...], (((1,), (1,)), ((), ())),
        preferred_element_type=jnp.float32) + b_ref[...]
    m = jnp.max(logits, axis=-1, keepdims=True)
    lm = logits - m
    e = jnp.exp(lm)
    s = jnp.sum(e, axis=-1, keepdims=True)
    inv_s = 1.0 / s
    p = e * inv_s
    probs_ref[...] = p

    iota = jax.lax.broadcasted_iota(jnp.int32, p.shape, 1)
    p1 = jnp.max(p, axis=-1, keepdims=True)
    i1 = jnp.min(jnp.where(p == p1, iota, NUM_EXPERTS), axis=-1, keepdims=True)
    pm = jnp.where(iota == i1, -1.0, p)
    p2 = jnp.max(pm, axis=-1, keepdims=True)
    i2 = jnp.min(jnp.where(pm == p2, iota, NUM_EXPERTS), axis=-1, keepdims=True)

    inv_d = 1.0 / (p1 + p2)
    p1n = p1 * inv_d
    p2n = p2 * inv_d
    idx_ref[...] = jnp.concatenate([i1, i2], axis=1)
    pk_ref[...] = jnp.concatenate([p1n, p2n], axis=1)

    usage_sum = jnp.sum(p, axis=0, keepdims=True)
    ent_sum = (jnp.sum(p * lm, keepdims=True).reshape(1, 1)
               - jnp.sum(jnp.log(s), keepdims=True).reshape(1, 1))
    conf_sum = jnp.sum(p1n, keepdims=True).reshape(1, 1)
    part_ref[...] = jnp.concatenate(
        [usage_sum, ent_sum, conf_sum, jnp.zeros((1, 62), jnp.float32)],
        axis=1).reshape(1, 1, 128)


def _stats_kernel(n_tokens, part_ref, usage_ref, loss_ref, ent_ref, conf_ref):
    part = part_ref[...].reshape(part_ref.shape[0], 128)
    inv_n = 1.0 / n_tokens
    u = jnp.sum(part[:, :NUM_EXPERTS], axis=0, keepdims=True) * inv_n
    usage_ref[...] = u
    t = 1.0 / NUM_EXPERTS
    kl = jnp.sum(t * (jnp.log(t) - jnp.log(u)), keepdims=True) / NUM_EXPERTS
    loss_ref[...] = kl.reshape(1, 1) * LOAD_BALANCE_WEIGHT
    ent_ref[...] = -(jnp.sum(part[:, NUM_EXPERTS:NUM_EXPERTS + 1],
                             keepdims=True).reshape(1, 1) * inv_n)
    conf_ref[...] = jnp.sum(part[:, NUM_EXPERTS + 1:NUM_EXPERTS + 2],
                            keepdims=True).reshape(1, 1) * inv_n


@jax.jit
def kernel(hidden_states, W, b):
    B, S, D = hidden_states.shape
    n = B * S
    x = hidden_states.reshape(n, D)
    b2 = b.reshape(1, NUM_EXPERTS)
    T = TOKEN_BLOCK
    nsteps = n // T

    idx, pk, probs, part = pl.pallas_call(
        _router_kernel,
        grid=(nsteps,),
        in_specs=[
            pl.BlockSpec((T, D), lambda i: (i, 0)),
            pl.BlockSpec((NUM_EXPERTS, D), lambda i: (0, 0)),
            pl.BlockSpec((1, NUM_EXPERTS), lambda i: (0, 0)),
        ],
        out_specs=(
            pl.BlockSpec((T, TOP_K), lambda i: (i, 0)),
            pl.BlockSpec((T, TOP_K), lambda i: (i, 0)),
            pl.BlockSpec((T, NUM_EXPERTS), lambda i: (i, 0)),
            pl.BlockSpec((1, 1, 128), lambda i: (i, 0, 0)),
        ),
        out_shape=(
            jax.ShapeDtypeStruct((n, TOP_K), jnp.int32),
            jax.ShapeDtypeStruct((n, TOP_K), jnp.float32),
            jax.ShapeDtypeStruct((n, NUM_EXPERTS), jnp.float32),
            jax.ShapeDtypeStruct((nsteps, 1, 128), jnp.float32),
        ),
        compiler_params=pltpu.CompilerParams(
            dimension_semantics=("parallel",)),
    )(x, W, b2)

    usage, loss, ent, conf = pl.pallas_call(
        functools.partial(_stats_kernel, float(n)),
        out_shape=(
            jax.ShapeDtypeStruct((1, NUM_EXPERTS), jnp.float32),
            jax.ShapeDtypeStruct((1, 1), jnp.float32),
            jax.ShapeDtypeStruct((1, 1), jnp.float32),
            jax.ShapeDtypeStruct((1, 1), jnp.float32),
        ),
    )(part)

    return (idx.reshape(B, S, TOP_K),
            pk.reshape(B, S, TOP_K),
            probs.reshape(B, S, NUM_EXPERTS),
            loss.reshape(()),
            ent.reshape(()),
            usage.reshape(NUM_EXPERTS),
            conf.reshape(()))


# speedup vs baseline: 1.0662x; 1.0621x over previous
import functools

import jax
import jax.numpy as jnp
from jax.experimental import pallas as pl
from jax.experimental.pallas import tpu as pltpu

EMBED_DIM = 768
NUM_EXPERTS = 64
TOP_K = 2
LOAD_BALANCE_WEIGHT = 0.01

TOKEN_BLOCK = 4096


def _router_kernel(x_ref, w_ref, b_ref, idx_ref, pk_ref, probs_ref, part_ref):
    logits = jax.lax.dot_general(
        x_ref[...], w_ref[...], (((1,), (1,)), ((), ())),
        preferred_element_type=jnp.float32) + b_ref[...]
    m = jnp.max(logits, axis=-1, keepdims=True)
    lm = logits - m
    e = jnp.exp(lm)
    s = jnp.sum(e, axis=-1, keepdims=True)
    inv_s = 1.0 / s
    p = e * inv_s
    probs_ref[...] = p.reshape(1, p.shape[0], NUM_EXPERTS)

    iota = jax.lax.broadcasted_iota(jnp.int32, p.shape, 1)
    p1 = jnp.max(p, axis=-1, keepdims=True)
    i1 = jnp.min(jnp.where(p == p1, iota, NUM_EXPERTS), axis=-1, keepdims=True)
    pm = jnp.where(iota == i1, -1.0, p)
    p2 = jnp.max(pm, axis=-1, keepdims=True)
    i2 = jnp.min(jnp.where(pm == p2, iota, NUM_EXPERTS), axis=-1, keepdims=True)

    inv_d = 1.0 / (p1 + p2)
    p1n = p1 * inv_d
    p2n = p2 * inv_d
    Tb = i1.shape[0]
    idx_ref[...] = jnp.concatenate([i1, i2], axis=1).reshape(1, Tb, TOP_K)
    pk_ref[...] = jnp.concatenate([p1n, p2n], axis=1).reshape(1, Tb, TOP_K)

    usage_sum = jnp.sum(p, axis=0, keepdims=True)
    ent_sum = (jnp.sum(p * lm, keepdims=True).reshape(1, 1)
               - jnp.sum(jnp.log(s), keepdims=True).reshape(1, 1))
    conf_sum = jnp.sum(p1n, keepdims=True).reshape(1, 1)
    part_ref[...] = jnp.concatenate(
        [usage_sum, ent_sum, conf_sum, jnp.zeros((1, 62), jnp.float32)],
        axis=1).reshape(1, 1, 128)


def _stats_kernel(n_tokens, part_ref, usage_ref, loss_ref, ent_ref, conf_ref):
    part = part_ref[...].reshape(part_ref.shape[0], 128)
    inv_n = 1.0 / n_tokens
    u = jnp.sum(part[:, :NUM_EXPERTS], axis=0, keepdims=True) * inv_n
    usage_ref[...] = u
    t = 1.0 / NUM_EXPERTS
    kl = jnp.sum(t * (jnp.log(t) - jnp.log(u)), keepdims=True) / NUM_EXPERTS
    loss_ref[...] = kl.reshape(1, 1) * LOAD_BALANCE_WEIGHT
    ent_ref[...] = -(jnp.sum(part[:, NUM_EXPERTS:NUM_EXPERTS + 1],
                             keepdims=True).reshape(1, 1) * inv_n)
    conf_ref[...] = jnp.sum(part[:, NUM_EXPERTS + 1:NUM_EXPERTS + 2],
                            keepdims=True).reshape(1, 1) * inv_n


@jax.jit
def kernel(hidden_states, W, b):
    B, S, D = hidden_states.shape
    n = B * S
    x = hidden_states.reshape(n, D)
    b2 = b.reshape(1, NUM_EXPERTS)
    T = TOKEN_BLOCK
    nsteps = n // T
    pr = S // T

    idx, pk, probs, part = pl.pallas_call(
        _router_kernel,
        grid=(nsteps,),
        in_specs=[
            pl.BlockSpec((T, D), lambda i: (i, 0)),
            pl.BlockSpec((NUM_EXPERTS, D), lambda i: (0, 0)),
            pl.BlockSpec((1, NUM_EXPERTS), lambda i: (0, 0)),
        ],
        out_specs=(
            pl.BlockSpec((1, T, TOP_K), lambda i: (i // pr, i % pr, 0)),
            pl.BlockSpec((1, T, TOP_K), lambda i: (i // pr, i % pr, 0)),
            pl.BlockSpec((1, T, NUM_EXPERTS), lambda i: (i // pr, i % pr, 0)),
            pl.BlockSpec((1, 1, 128), lambda i: (i, 0, 0)),
        ),
        out_shape=(
            jax.ShapeDtypeStruct((B, S, TOP_K), jnp.int32),
            jax.ShapeDtypeStruct((B, S, TOP_K), jnp.float32),
            jax.ShapeDtypeStruct((B, S, NUM_EXPERTS), jnp.float32),
            jax.ShapeDtypeStruct((nsteps, 1, 128), jnp.float32),
        ),
        compiler_params=pltpu.CompilerParams(
            dimension_semantics=("parallel",)),
    )(x, W, b2)

    usage, loss, ent, conf = pl.pallas_call(
        functools.partial(_stats_kernel, float(n)),
        out_shape=(
            jax.ShapeDtypeStruct((1, NUM_EXPERTS), jnp.float32),
            jax.ShapeDtypeStruct((1, 1), jnp.float32),
            jax.ShapeDtypeStruct((1, 1), jnp.float32),
            jax.ShapeDtypeStruct((1, 1), jnp.float32),
        ),
    )(part)

    return (idx,
            pk,
            probs,
            loss.reshape(()),
            ent.reshape(()),
            usage.reshape(NUM_EXPERTS),
            conf.reshape(()))
